# baseline (speedup 1.0000x reference)
import jax
import jax.numpy as jnp
from jax.experimental import pallas as pl
from jax.experimental.pallas import tpu as pltpu

_ATTRS = 12
_T = 512
_D = 32
_W = _ATTRS + _T * _D
_BBLK = 16


def _cumsum_lanes(a):
    n = a.shape[-1]
    z = jnp.zeros_like(a)
    k = 1
    while k < n:
        shifted = jnp.concatenate([z[:, :k], a[:, :-k]], axis=-1)
        a = a + shifted
        k *= 2
    return a


def _body(x_ref, o_ref):
    x = x_ref[...]
    feat = x[:, _ATTRS:]
    b = feat.shape[0]
    f3 = feat.reshape(b, _T, _D)
    last = f3[:, :, _D - 1]
    s = _cumsum_lanes(last)
    eq = (s == 1.0).astype(jnp.float32)
    cnt = _cumsum_lanes(eq)
    keep = jnp.where((eq == 0.0) | (cnt == 1.0), 1.0, 0.0)
    m3 = jnp.broadcast_to(keep[:, :, None], (b, _T, _D)).reshape(b, _T * _D)
    o_ref[...] = jnp.concatenate([x[:, :_ATTRS], feat * m3], axis=1)


def kernel(input):
    B = input.shape[0]
    bblk = _BBLK if B % _BBLK == 0 else B
    grid = (B // bblk,)
    return pl.pallas_call(
        _body,
        grid=grid,
        in_specs=[pl.BlockSpec((bblk, _W), lambda i: (i, 0))],
        out_specs=pl.BlockSpec((bblk, _W), lambda i: (i, 0)),
        out_shape=jax.ShapeDtypeStruct((B, _W), jnp.float32),
    )(input)


# speedup vs baseline: 15.7472x; 15.7472x over previous
import numpy as np
import jax
import jax.numpy as jnp
from jax.experimental import pallas as pl
from jax.experimental.pallas import tpu as pltpu

_ATTRS = 12
_T = 512
_D = 32
_F = _T * _D
_W = _ATTRS + _F
_BBLK = 64


def _make_sigma():
    arr = -np.ones(_F, dtype=np.int64)
    for t in range(_T):
        arr[_D - 1 + _D * t] = t
    w = _F
    for k in range(1, 6):
        w2 = w // 2
        low, up = arr[:w2], arr[w2:w]
        r = 2 ** (5 - k)
        p = 2 ** (6 - k)
        j = np.arange(w2)
        arr = np.where((j % p) >= p // 2, low, np.roll(up, -r))
        w = w2
    return arr[:_T]


_SIGMA = _make_sigma()
_U = (_SIGMA[:, None] <= _SIGMA[None, :]).astype(np.float32)
_E = np.zeros((_T + 1, _W), dtype=np.float32)
for _p in range(_T):
    _E[_p, _ATTRS + _D * _SIGMA[_p]: _ATTRS + _D * _SIGMA[_p] + _D] = 1.0
_E[_T, :_ATTRS] = 1.0


def _body(x_ref, u_ref, e_ref, o_ref):
    x = x_ref[...]
    a = x[:, _ATTRS:_W]
    b = a.shape[0]
    w = _F
    for k in range(1, 6):
        w2 = w // 2
        low, up = a[:, :w2], a[:, w2:w]
        r = 2 ** (5 - k)
        p = 2 ** (6 - k)
        j = jax.lax.broadcasted_iota(jnp.int32, (b, w2), 1)
        a = jnp.where((j % p) >= p // 2, low, pltpu.roll(up, w2 - r, 1))
        w = w2
    vperm = a[:, :_T]
    u = u_ref[...]
    s = jax.lax.dot(vperm, u, preferred_element_type=jnp.float32)
    eqf = (s == 1.0).astype(jnp.float32)
    cnt = jax.lax.dot(eqf, u, preferred_element_type=jnp.float32)
    keep = jnp.where((eqf == 0.0) | (cnt == 1.0), 1.0, 0.0)
    keepa = jnp.concatenate([keep, jnp.ones((b, 1), jnp.float32)], axis=1)
    mask_full = jax.lax.dot(
        keepa.astype(jnp.bfloat16), e_ref[...],
        preferred_element_type=jnp.float32)
    o_ref[...] = x * mask_full


def kernel(input):
    B = input.shape[0]
    bblk = _BBLK if B % _BBLK == 0 else B
    u = jnp.asarray(_U)
    e = jnp.asarray(_E).astype(jnp.bfloat16)
    grid = (B // bblk,)
    return pl.pallas_call(
        _body,
        grid=grid,
        in_specs=[
            pl.BlockSpec((bblk, _W), lambda i: (i, 0)),
            pl.BlockSpec((_T, _T), lambda i: (0, 0)),
            pl.BlockSpec((_T + 1, _W), lambda i: (0, 0)),
        ],
        out_specs=pl.BlockSpec((bblk, _W), lambda i: (i, 0)),
        out_shape=jax.ShapeDtypeStruct((B, _W), jnp.float32),
    )(input, u, e)


# speedup vs baseline: 18.8018x; 1.1940x over previous
import numpy as np
import jax
import jax.numpy as jnp
from jax.experimental import pallas as pl
from jax.experimental.pallas import tpu as pltpu

_ATTRS = 12
_T = 512
_D = 32
_F = _T * _D
_W = _ATTRS + _F
_BBLK = 128


def _make_sigma():
    arr = -np.ones(_F, dtype=np.int64)
    for t in range(_T):
        arr[_D - 1 + _D * t] = t
    w = _F
    for k in range(1, 6):
        w2 = w // 2
        low, up = arr[:w2], arr[w2:w]
        r = 2 ** (5 - k)
        p = 2 ** (6 - k)
        j = np.arange(w2)
        arr = np.where((j % p) >= p // 2, low, np.roll(up, -r))
        w = w2
    return arr[:_T]


_SIGMA = _make_sigma()
_U = (_SIGMA[:, None] <= _SIGMA[None, :]).astype(np.float32)
_E = np.zeros((_T + 1, _W), dtype=np.float32)
for _p in range(_T):
    _E[_p, _ATTRS + _D * _SIGMA[_p]: _ATTRS + _D * _SIGMA[_p] + _D] = 1.0
_E[_T, :_ATTRS] = 1.0


def _body(x_ref, u_ref, e_ref, o_ref):
    x = x_ref[...]
    a = x[:, _ATTRS:_W]
    b = a.shape[0]
    w = _F
    for k in range(1, 6):
        w2 = w // 2
        low, up = a[:, :w2], a[:, w2:w]
        r = 2 ** (5 - k)
        p = 2 ** (6 - k)
        j = jax.lax.broadcasted_iota(jnp.int32, (b, w2), 1)
        a = jnp.where((j % p) >= p // 2, low, pltpu.roll(up, w2 - r, 1))
        w = w2
    vperm = a[:, :_T]
    u = u_ref[...]
    s = jax.lax.dot(vperm, u, preferred_element_type=jnp.float32)
    eqf = (s == 1.0).astype(jnp.float32)
    cnt = jax.lax.dot(eqf, u, preferred_element_type=jnp.float32)
    keep = jnp.where((eqf == 0.0) | (cnt == 1.0), 1.0, 0.0)
    keepa = jnp.concatenate([keep, jnp.ones((b, 1), jnp.float32)], axis=1)
    mask_full = jax.lax.dot(
        keepa.astype(jnp.bfloat16), e_ref[...],
        preferred_element_type=jnp.float32)
    o_ref[...] = x * mask_full


def kernel(input):
    B = input.shape[0]
    bblk = _BBLK if B % _BBLK == 0 else B
    u = jnp.asarray(_U)
    e = jnp.asarray(_E).astype(jnp.bfloat16)
    grid = (B // bblk,)
    return pl.pallas_call(
        _body,
        grid=grid,
        in_specs=[
            pl.BlockSpec((bblk, _W), lambda i: (i, 0)),
            pl.BlockSpec((_T, _T), lambda i: (0, 0)),
            pl.BlockSpec((_T + 1, _W), lambda i: (0, 0)),
        ],
        out_specs=pl.BlockSpec((bblk, _W), lambda i: (i, 0)),
        out_shape=jax.ShapeDtypeStruct((B, _W), jnp.float32),
    )(input, u, e)


# speedup vs baseline: 19.3980x; 1.0317x over previous
import numpy as np
import jax
import jax.numpy as jnp
from jax.experimental import pallas as pl
from jax.experimental.pallas import tpu as pltpu

_ATTRS = 12
_T = 512
_D = 32
_F = _T * _D
_W = _ATTRS + _F
_BBLK = 128
_GLANE = 11


def _fold_plan():
    plan = []
    rho = (_ATTRS + _D - 1) % _D
    for k in range(1, 6):
        p = 2 ** (6 - k)
        r = p // 2
        plan.append((p, (rho % p) >= r))
        rho = rho % r
    return plan


_PLAN = _fold_plan()


def _make_sigma():
    arr = -np.ones(_F, dtype=np.int64)
    for t in range(_T - 1):
        arr[_ATTRS + _D - 1 + _D * t] = t
    w = _F
    for p, low_upper in _PLAN:
        w2 = w // 2
        r = p // 2
        low, up = arr[:w2], arr[w2:w]
        j = np.arange(w2)
        sel_low = ((j % p) >= r) if low_upper else ((j % p) < r)
        arr = np.where(sel_low, low, np.roll(up, -r))
        w = w2
    sigma = arr[:_T]
    assert sigma[_GLANE] == -1
    sigma[_GLANE] = _T - 1
    assert sorted(sigma.tolist()) == list(range(_T))
    return sigma


_SIGMA = _make_sigma()
_U = (_SIGMA[:, None] <= _SIGMA[None, :]).astype(np.float32)
_E = np.zeros((_T + 1, _W), dtype=np.float32)
for _p in range(_T):
    _E[_p, _ATTRS + _D * _SIGMA[_p]: _ATTRS + _D * _SIGMA[_p] + _D] = 1.0
_E[_T, :_ATTRS] = 1.0


def _body(x_ref, u_ref, e_ref, o_ref):
    x = x_ref[...]
    a = x[:, :_F]
    b = a.shape[0]
    w = _F
    for p, low_upper in _PLAN:
        w2 = w // 2
        r = p // 2
        low, up = a[:, :w2], a[:, w2:w]
        j = jax.lax.broadcasted_iota(jnp.int32, (b, w2), 1)
        sel = ((j % p) >= r) if low_upper else ((j % p) < r)
        a = jnp.where(sel, low, pltpu.roll(up, w2 - r, 1))
        w = w2
    lane = jax.lax.broadcasted_iota(jnp.int32, (b, _T), 1)
    t511 = x[:, _W - 1:_W]
    vperm = jnp.where(lane == _GLANE, t511, a[:, :_T])
    u = u_ref[...]
    s = jax.lax.dot(vperm, u, preferred_element_type=jnp.float32)
    eqf = (s == 1.0).astype(jnp.float32)
    cnt = jax.lax.dot(eqf, u, preferred_element_type=jnp.float32)
    keep = jnp.where((eqf == 0.0) | (cnt == 1.0), 1.0, 0.0)
    keepa = jnp.concatenate([keep, jnp.ones((b, 1), jnp.float32)], axis=1)
    mask_full = jax.lax.dot(
        keepa.astype(jnp.bfloat16), e_ref[...],
        preferred_element_type=jnp.float32)
    o_ref[...] = x * mask_full


def kernel(input):
    B = input.shape[0]
    bblk = _BBLK if B % _BBLK == 0 else B
    u = jnp.asarray(_U)
    e = jnp.asarray(_E).astype(jnp.bfloat16)
    grid = (B // bblk,)
    return pl.pallas_call(
        _body,
        grid=grid,
        in_specs=[
            pl.BlockSpec((bblk, _W), lambda i: (i, 0)),
            pl.BlockSpec((_T, _T), lambda i: (0, 0)),
            pl.BlockSpec((_T + 1, _W), lambda i: (0, 0)),
        ],
        out_specs=pl.BlockSpec((bblk, _W), lambda i: (i, 0)),
        out_shape=jax.ShapeDtypeStruct((B, _W), jnp.float32),
    )(input, u, e)


# speedup vs baseline: 19.4691x; 1.0037x over previous
import numpy as np
import jax
import jax.numpy as jnp
from jax.experimental import pallas as pl
from jax.experimental.pallas import tpu as pltpu

_ATTRS = 12
_T = 512
_D = 32
_F = _T * _D
_W = _ATTRS + _F
_BBLK = 128
_GLANE = 11


def _fold_plan():
    plan = []
    rho = (_ATTRS + _D - 1) % _D
    for k in range(1, 6):
        p = 2 ** (6 - k)
        r = p // 2
        plan.append((p, (rho % p) >= r))
        rho = rho % r
    return plan


_PLAN = _fold_plan()


def _make_sigma():
    arr = -np.ones(_F, dtype=np.int64)
    for t in range(_T - 1):
        arr[_ATTRS + _D - 1 + _D * t] = t
    w = _F
    for p, low_upper in _PLAN:
        w2 = w // 2
        r = p // 2
        low, up = arr[:w2], arr[w2:w]
        j = np.arange(w2)
        sel_low = ((j % p) >= r) if low_upper else ((j % p) < r)
        arr = np.where(sel_low, low, np.roll(up, -r))
        w = w2
    sigma = arr[:_T]
    assert sigma[_GLANE] == -1
    sigma[_GLANE] = _T - 1
    assert sorted(sigma.tolist()) == list(range(_T))
    return sigma


_SIGMA = _make_sigma()
_U = (_SIGMA[:, None] <= _SIGMA[None, :]).astype(np.float32)
_E = np.zeros((_T + 1, _W), dtype=np.float32)
for _p in range(_T):
    _E[_p, _ATTRS + _D * _SIGMA[_p]: _ATTRS + _D * _SIGMA[_p] + _D] = 1.0
_E[_T, :_ATTRS] = 1.0


def _body(x_ref, u_ref, e_ref, o_ref):
    x = x_ref[...]
    a = x[:, :_F]
    b = a.shape[0]
    w = _F
    for p, low_upper in _PLAN:
        w2 = w // 2
        r = p // 2
        low, up = a[:, :w2], a[:, w2:w]
        j = jax.lax.broadcasted_iota(jnp.int32, (b, w2), 1)
        sel = ((j % p) >= r) if low_upper else ((j % p) < r)
        a = jnp.where(sel, low, pltpu.roll(up, w2 - r, 1))
        w = w2
    lane = jax.lax.broadcasted_iota(jnp.int32, (b, _T), 1)
    t511 = x[:, _W - 1:_W]
    vperm = jnp.where(lane == _GLANE, t511, a[:, :_T])
    u = u_ref[...]
    s = jax.lax.dot(vperm, u, preferred_element_type=jnp.float32)
    eqf = (s == 1.0).astype(jnp.float32)
    cnt = jax.lax.dot(eqf, u, preferred_element_type=jnp.float32)
    keep = jnp.where((eqf == 0.0) | (cnt == 1.0), 1.0, 0.0)
    keepa = jnp.concatenate([keep, jnp.ones((b, 1), jnp.float32)], axis=1)
    mask_full = jax.lax.dot(
        keepa.astype(jnp.int8), e_ref[...],
        preferred_element_type=jnp.int32)
    o_ref[...] = x * mask_full.astype(jnp.float32)


def kernel(input):
    B = input.shape[0]
    bblk = _BBLK if B % _BBLK == 0 else B
    u = jnp.asarray(_U)
    e = jnp.asarray(_E).astype(jnp.int8)
    grid = (B // bblk,)
    return pl.pallas_call(
        _body,
        grid=grid,
        in_specs=[
            pl.BlockSpec((bblk, _W), lambda i: (i, 0)),
            pl.BlockSpec((_T, _T), lambda i: (0, 0)),
            pl.BlockSpec((_T + 1, _W), lambda i: (0, 0)),
        ],
        out_specs=pl.BlockSpec((bblk, _W), lambda i: (i, 0)),
        out_shape=jax.ShapeDtypeStruct((B, _W), jnp.float32),
    )(input, u, e)
